# Initial kernel scaffold; baseline (speedup 1.0000x reference)
import jax
import jax.numpy as jnp
from jax.experimental import pallas as pl


def kernel(input, weight):
    raise NotImplementedError("write your pallas kernel here")


# speedup vs baseline: 5.0827x; 5.0827x over previous
import functools

import jax
import jax.numpy as jnp
from jax import lax
from jax.experimental import pallas as pl
from jax.experimental.pallas import tpu as pltpu
from jax.experimental.pallas import tpu_sc as plsc

N_D = 64
LANES = 16
CHUNK = 512
CHUNK2 = CHUNK // 2


def _sc_embed(w_flat, idx_flat, n_rows):
    info = plsc.get_sparse_core_info()
    num_workers = info.num_cores * info.num_subcores
    rows_per_w = n_rows // num_workers
    n_chunks = rows_per_w // CHUNK
    nc2 = n_chunks // 2
    mesh = plsc.VectorSubcoreMesh(core_axis_name="c", subcore_axis_name="s")

    @functools.partial(
        pl.kernel,
        mesh=mesh,
        out_type=jax.ShapeDtypeStruct((n_rows // 2, 2 * N_D), jnp.float32),
        scratch_types=[
            pltpu.VMEM((2 * N_D,), jnp.float32),
            pltpu.VMEM((CHUNK,), jnp.int32),
            pltpu.VMEM((CHUNK,), jnp.int32),
            pltpu.VMEM((CHUNK2, 2 * N_D), jnp.float32),
            pltpu.VMEM((CHUNK2, 2 * N_D), jnp.float32),
            pltpu.SemaphoreType.DMA,
            pltpu.SemaphoreType.DMA,
            pltpu.SemaphoreType.DMA,
            pltpu.SemaphoreType.DMA,
        ],
    )
    def k(w_hbm, idx_hbm, out_hbm, w_v, idx0, idx1, rows0, rows1,
          semi0, semi1, semo0, semo1):
        wid = lax.axis_index("s") * info.num_cores + lax.axis_index("c")
        base = wid * rows_per_w

        pltpu.sync_copy(w_hbm, w_v)
        w0 = [w_v[pl.ds(j * LANES, LANES)] for j in range(N_D // LANES)]
        dif = [w_v[pl.ds(N_D + j * LANES, LANES)] - w0[j]
               for j in range(N_D // LANES)]

        def compute(idx_v, rows_v):
            def group_body(g, carry):
                gbase = g * LANES
                fv = idx_v[pl.ds(gbase, LANES)].astype(jnp.float32)
                line = g * (LANES // 2)
                for u in range(LANES):
                    fi = fv[u]
                    col = (u % 2) * N_D
                    for j in range(N_D // LANES):
                        rows_v[line + u // 2,
                               pl.ds(col + j * LANES, LANES)] = (
                            w0[j] + fi * dif[j])
                return carry
            lax.fori_loop(0, CHUNK // LANES, group_body, 0)

        bufs = ((idx0, rows0, semi0, semo0), (idx1, rows1, semi1, semo1))

        for b, (idx_v, _, semi, _) in enumerate(bufs):
            off = pl.multiple_of(base + b * CHUNK, CHUNK)
            pltpu.async_copy(idx_hbm.at[pl.ds(off, CHUNK)], idx_v, semi)

        def outer(i2, carry):
            for b, (idx_v, rows_v, semi, semo) in enumerate(bufs):
                i = 2 * i2 + b
                off = pl.multiple_of(base + i * CHUNK, CHUNK)
                off2 = pl.multiple_of((base + i * CHUNK) // 2, CHUNK2)
                pltpu.make_async_copy(
                    idx_hbm.at[pl.ds(off, CHUNK)], idx_v, semi).wait()

                @pl.when(i2 > 0)
                def _drain():
                    pltpu.make_async_copy(
                        rows_v, out_hbm.at[pl.ds(off2, CHUNK2)], semo).wait()

                compute(idx_v, rows_v)
                pltpu.async_copy(rows_v, out_hbm.at[pl.ds(off2, CHUNK2)], semo)

                @pl.when(i2 < nc2 - 1)
                def _prefetch():
                    offn = pl.multiple_of(base + (i + 2) * CHUNK, CHUNK)
                    pltpu.async_copy(
                        idx_hbm.at[pl.ds(offn, CHUNK)], idx_v, semi)
            return carry

        lax.fori_loop(0, nc2, outer, 0)

        for b, (_, rows_v, _, semo) in enumerate(bufs):
            off2 = pl.multiple_of(
                (base + (n_chunks - 2 + b) * CHUNK) // 2, CHUNK2)
            pltpu.make_async_copy(
                rows_v, out_hbm.at[pl.ds(off2, CHUNK2)], semo).wait()

    return k(w_flat, idx_flat)


def kernel(input, weight):
    b, h = input.shape
    n_rows = b * h
    out = _sc_embed(weight.reshape(2 * N_D), input.reshape(n_rows), n_rows)
    return out.reshape(b, h, N_D)


# speedup vs baseline: 6.6981x; 1.3178x over previous
import functools

import jax
import jax.numpy as jnp
from jax import lax
from jax.experimental import pallas as pl
from jax.experimental.pallas import tpu as pltpu
from jax.experimental.pallas import tpu_sc as plsc

N_D = 64
LANES = 16
NB = 2
HIST_FULL = 192
ROWS = 200 * NB


def _sc_embed(w_flat, idx_flat, n_batch, hist):
    info = plsc.get_sparse_core_info()
    num_workers = info.num_cores * info.num_subcores
    bat_per_w = n_batch // num_workers
    n_chunks = bat_per_w // NB
    nc2 = n_chunks // 2
    mesh = plsc.VectorSubcoreMesh(core_axis_name="c", subcore_axis_name="s")

    @functools.partial(
        pl.kernel,
        mesh=mesh,
        out_type=jax.ShapeDtypeStruct((n_batch, hist, N_D), jnp.float32),
        scratch_types=[
            pltpu.VMEM((2 * N_D,), jnp.float32),
            pltpu.VMEM((ROWS,), jnp.int32),
            pltpu.VMEM((ROWS,), jnp.int32),
            pltpu.VMEM((NB, hist, N_D), jnp.float32),
            pltpu.VMEM((NB, hist, N_D), jnp.float32),
            pltpu.SemaphoreType.DMA,
            pltpu.SemaphoreType.DMA,
            pltpu.SemaphoreType.DMA,
            pltpu.SemaphoreType.DMA,
        ],
    )
    def k(w_hbm, idx_hbm, out_hbm, w_v, idx0, idx1, rows0, rows1,
          semi0, semi1, semo0, semo1):
        wid = lax.axis_index("s") * info.num_cores + lax.axis_index("c")
        base = wid * bat_per_w * hist

        pltpu.sync_copy(w_hbm, w_v)
        w0 = [w_v[pl.ds(j * LANES, LANES)] for j in range(N_D // LANES)]
        dif = [w_v[pl.ds(N_D + j * LANES, LANES)] - w0[j]
               for j in range(N_D // LANES)]

        def emit(rows_v, bb, h, fv, lo):
            for u in range(lo, LANES):
                fi = fv[u]
                for j in range(N_D // LANES):
                    rows_v[bb, h + u, pl.ds(j * LANES, LANES)] = (
                        w0[j] + fi * dif[j])

        def compute(idx_v, rows_v):
            for bb in range(NB):
                def group_body(g, carry, bb=bb):
                    h = g * LANES
                    fv = idx_v[pl.ds(bb * hist + h, LANES)].astype(
                        jnp.float32)
                    emit(rows_v, bb, h, fv, 0)
                    return carry
                lax.fori_loop(0, HIST_FULL // LANES, group_body, 0)
                tl = hist - LANES
                fv = idx_v[pl.ds(bb * hist + tl, LANES)].astype(jnp.float32)
                emit(rows_v, bb, tl, fv, HIST_FULL - tl)

        bufs = ((idx0, rows0, semi0, semo0), (idx1, rows1, semi1, semo1))

        for b, (idx_v, _, semi, _) in enumerate(bufs):
            off = pl.multiple_of(base + b * ROWS, ROWS)
            pltpu.async_copy(idx_hbm.at[pl.ds(off, ROWS)], idx_v, semi)

        def outer(i2, carry):
            for b, (idx_v, rows_v, semi, semo) in enumerate(bufs):
                i = 2 * i2 + b
                off = pl.multiple_of(base + i * ROWS, ROWS)
                boff = pl.multiple_of(wid * bat_per_w + i * NB, NB)
                pltpu.make_async_copy(
                    idx_hbm.at[pl.ds(off, ROWS)], idx_v, semi).wait()

                @pl.when(i2 > 0)
                def _drain():
                    pltpu.make_async_copy(
                        rows_v, out_hbm.at[pl.ds(boff, NB)], semo).wait()

                compute(idx_v, rows_v)
                pltpu.async_copy(rows_v, out_hbm.at[pl.ds(boff, NB)], semo)

                @pl.when(i2 < nc2 - 1)
                def _prefetch():
                    offn = pl.multiple_of(base + (i + 2) * ROWS, ROWS)
                    pltpu.async_copy(
                        idx_hbm.at[pl.ds(offn, ROWS)], idx_v, semi)
            return carry

        lax.fori_loop(0, nc2, outer, 0)

        for b, (_, rows_v, _, semo) in enumerate(bufs):
            boff = pl.multiple_of(
                wid * bat_per_w + (n_chunks - 2 + b) * NB, NB)
            pltpu.make_async_copy(
                rows_v, out_hbm.at[pl.ds(boff, NB)], semo).wait()

    return k(w_flat, idx_flat)


def kernel(input, weight):
    b, h = input.shape
    return _sc_embed(weight.reshape(2 * N_D), input.reshape(b * h), b, h)


# speedup vs baseline: 31.3376x; 4.6786x over previous
import functools

import jax
import jax.numpy as jnp
from jax import lax
from jax.experimental import pallas as pl
from jax.experimental.pallas import tpu as pltpu
from jax.experimental.pallas import tpu_sc as plsc

N_D = 64
LANES = 16
HCH = 4


def _sc_embed(w_flat, idx_t, hist, n_batch):
    info = plsc.get_sparse_core_info()
    num_workers = info.num_cores * info.num_subcores
    bpw = n_batch // num_workers
    n_chunks = hist // HCH
    nc2 = n_chunks // 2
    nvec = bpw // LANES
    mesh = plsc.VectorSubcoreMesh(core_axis_name="c", subcore_axis_name="s")

    @functools.partial(
        pl.kernel,
        mesh=mesh,
        out_type=jax.ShapeDtypeStruct((hist, N_D, n_batch), jnp.float32),
        scratch_types=[
            pltpu.VMEM((2 * N_D,), jnp.float32),
            pltpu.VMEM((hist, bpw), jnp.int32),
            pltpu.VMEM((HCH, N_D, bpw), jnp.float32),
            pltpu.VMEM((HCH, N_D, bpw), jnp.float32),
            pltpu.SemaphoreType.DMA,
            pltpu.SemaphoreType.DMA,
        ],
    )
    def k(w_hbm, idx_hbm, out_hbm, w_v, idx_v, rows0, rows1, semo0, semo1):
        wid = lax.axis_index("s") * info.num_cores + lax.axis_index("c")
        b0 = pl.multiple_of(wid * bpw, bpw)

        pltpu.sync_copy(idx_hbm.at[pl.ds(0, hist), pl.ds(b0, bpw)], idx_v)
        pltpu.sync_copy(w_hbm, w_v)
        wv0 = [w_v[pl.ds(j * LANES, LANES)] for j in range(N_D // LANES)]
        wv1 = [w_v[pl.ds(N_D + j * LANES, LANES)] for j in range(N_D // LANES)]
        w0s = [wv0[j][l] for j in range(N_D // LANES) for l in range(LANES)]
        difs = [wv1[j][l] - w0s[j * LANES + l]
                for j in range(N_D // LANES) for l in range(LANES)]

        def compute(i, rows_v):
            def h_body(hh, carry):
                h = i * HCH + hh
                fv = [idx_v[h, pl.ds(c * LANES, LANES)].astype(jnp.float32)
                      for c in range(nvec)]
                for d in range(N_D):
                    for c in range(nvec):
                        rows_v[hh, d, pl.ds(c * LANES, LANES)] = (
                            w0s[d] + fv[c] * difs[d])
                return carry
            lax.fori_loop(0, HCH, h_body, 0)

        bufs = ((rows0, semo0), (rows1, semo1))

        def outer(i2, carry):
            for b, (rows_v, semo) in enumerate(bufs):
                i = 2 * i2 + b
                hoff = pl.multiple_of(i * HCH, HCH)

                @pl.when(i2 > 0)
                def _drain():
                    pltpu.make_async_copy(
                        rows_v,
                        out_hbm.at[pl.ds(hoff, HCH), pl.ds(0, N_D),
                                   pl.ds(b0, bpw)],
                        semo).wait()

                compute(i, rows_v)
                pltpu.async_copy(
                    rows_v,
                    out_hbm.at[pl.ds(hoff, HCH), pl.ds(0, N_D),
                               pl.ds(b0, bpw)],
                    semo)
            return carry

        lax.fori_loop(0, nc2, outer, 0)

        for b, (rows_v, semo) in enumerate(bufs):
            hoff = pl.multiple_of((n_chunks - 2 + b) * HCH, HCH)
            pltpu.make_async_copy(
                rows_v,
                out_hbm.at[pl.ds(hoff, HCH), pl.ds(0, N_D), pl.ds(b0, bpw)],
                semo).wait()

    return k(w_flat, idx_t)


def kernel(input, weight):
    b, h = input.shape
    out_t = _sc_embed(weight.reshape(2 * N_D), input.T, h, b)
    return jnp.transpose(out_t, (2, 0, 1))
